# baseline (speedup 1.0000x reference)
import functools

import jax
import jax.numpy as jnp
from jax import lax
from jax.experimental import pallas as pl
from jax.experimental.pallas import tpu as pltpu
from jax.experimental.pallas import tpu_sc as plsc

N = 10000
D = 128
E = 320000
EPS_BN = 1e-5

NC = 2
NS = 16
NW = NC * NS
CHUNK = 128
CPW = 80
EP = NW * CHUNK * CPW
NREAL = E // CHUNK
S = 16
NST = CPW // S
RPS = S // 2
RPT = 624
TAIL = N - NS * RPT
XSPLIT = 4992

_sc_mesh = plsc.VectorSubcoreMesh(core_axis_name="c", subcore_axis_name="s")


@functools.partial(
    pl.kernel,
    mesh=_sc_mesh,
    out_type=jax.ShapeDtypeStruct((2 * N, D), jnp.float32),
    scratch_types=[
        pltpu.VMEM_SHARED((N, D), jnp.float32),
        pltpu.VMEM((S, CHUNK), jnp.int32),
        pltpu.VMEM((S, CHUNK), jnp.int32),
        pltpu.VMEM((S, CHUNK), jnp.int32),
        pltpu.VMEM((S, CHUNK), jnp.int32),
        pltpu.VMEM((CHUNK, D), jnp.float32),
        pltpu.VMEM((CHUNK, D), jnp.float32),
        pltpu.SemaphoreType.DMA,
        pltpu.SemaphoreType.DMA,
        pltpu.SemaphoreType.DMA,
        pltpu.SemaphoreType.DMA,
        pltpu.SemaphoreType.DMA,
        pltpu.SemaphoreType.DMA,
    ],
)
def _sc_agg(x_hbm, src_hbm, dst_hbm, zero_hbm, out_hbm, acc,
            si0, di0, si1, di1, rb0, rb1, gs0, gs1, ss0, ss1, isrc, idst):
    c = lax.axis_index("c")
    s_ax = lax.axis_index("s")
    wid = c * NS + s_ax
    base = wid * CPW
    row0 = s_ax * RPT

    mine = jnp.where(c == 0, row0 < XSPLIT, row0 >= XSPLIT)

    @pl.when(mine)
    def _():
        pltpu.sync_copy(x_hbm.at[pl.ds(row0, RPT)], acc.at[pl.ds(row0, RPT)])

    @pl.when(jnp.logical_not(mine))
    def _():
        pltpu.sync_copy(zero_hbm.at[pl.ds(0, RPT)], acc.at[pl.ds(row0, RPT)])

    @pl.when(s_ax == NS - 1)
    def _():
        tmine = jnp.where(c == 0, NS * RPT < XSPLIT, NS * RPT >= XSPLIT)

        @pl.when(tmine)
        def _():
            pltpu.sync_copy(x_hbm.at[pl.ds(NS * RPT, TAIL)],
                            acc.at[pl.ds(NS * RPT, TAIL)])

        @pl.when(jnp.logical_not(tmine))
        def _():
            pltpu.sync_copy(zero_hbm.at[pl.ds(0, TAIL)],
                            acc.at[pl.ds(NS * RPT, TAIL)])

    plsc.subcore_barrier()

    sbuf = (si0, si1)
    dbuf = (di0, di1)
    pltpu.async_copy(src_hbm.at[pl.ds(base, S)], si0, isrc)
    pltpu.async_copy(dst_hbm.at[pl.ds(base, S)], di0, idst)

    for st in range(NST):
        p = st % 2
        sidx = sbuf[p]
        didx = dbuf[p]
        pltpu.make_async_copy(src_hbm.at[pl.ds(base + st * S, S)], sidx, isrc).wait()
        pltpu.make_async_copy(dst_hbm.at[pl.ds(base + st * S, S)], didx, idst).wait()

        def body(j, carry, st=st, sidx=sidx, didx=didx):
            g0 = base + st * S + 2 * j
            g1 = g0 + 1

            def slot(i_loc, g, rb, gsem, ssem):
                first = (st == 0) & (j == 0) if st == 0 else jnp.bool_(False)

                @pl.when(jnp.logical_and(jnp.logical_not(first), g - 2 < NREAL))
                def _():
                    pltpu.make_async_copy(rb, acc.at[didx.at[i_loc]], ssem).wait()

                @pl.when(g < NREAL)
                def _():
                    pltpu.async_copy(x_hbm.at[sidx.at[i_loc]], rb, gsem)

            def fire(i_loc, g, rb, gsem, ssem):
                @pl.when(g < NREAL)
                def _():
                    pltpu.make_async_copy(x_hbm.at[sidx.at[i_loc]], rb, gsem).wait()
                    pltpu.async_copy(rb, acc.at[didx.at[i_loc]], ssem, add=True)

            slot(2 * j, g0, rb0, gs0, ss0)
            slot(2 * j + 1, g1, rb1, gs1, ss1)
            fire(2 * j, g0, rb0, gs0, ss0)
            fire(2 * j + 1, g1, rb1, gs1, ss1)

            if st < NST - 1:
                @pl.when(j == 1)
                def _():
                    nxt = base + (st + 1) * S
                    pltpu.async_copy(src_hbm.at[pl.ds(nxt, S)], sbuf[1 - p], isrc)
                    pltpu.async_copy(dst_hbm.at[pl.ds(nxt, S)], dbuf[1 - p], idst)

            return carry

        lax.fori_loop(0, RPS, body, 0)

    last0 = base + 2 * (CPW // 2 - 1)

    @pl.when(last0 < NREAL)
    def _():
        pltpu.make_async_copy(rb0, acc.at[dbuf[(NST - 1) % 2].at[0]], ss0).wait()

    @pl.when(last0 + 1 < NREAL)
    def _():
        pltpu.make_async_copy(rb1, acc.at[dbuf[(NST - 1) % 2].at[1]], ss1).wait()

    plsc.subcore_barrier()
    pltpu.sync_copy(acc.at[pl.ds(row0, RPT)],
                    out_hbm.at[pl.ds(c * N + row0, RPT)])

    @pl.when(s_ax == NS - 1)
    def _():
        pltpu.sync_copy(acc.at[pl.ds(NS * RPT, TAIL)],
                        out_hbm.at[pl.ds(c * N + NS * RPT, TAIL)])


BR = 1000
NB = N // BR


def _tc_mlp_body(p_ref, wa_ref, ba_ref, wb_ref, bb_ref, g_ref, be_ref, o_ref,
                 z_sc, st_sc):
    ph = pl.program_id(0)
    b = pl.program_id(1)
    row = pl.multiple_of(b * BR, 8)

    @pl.when(ph == 0)
    def _():
        @pl.when(b == 0)
        def _():
            st_sc[...] = jnp.zeros_like(st_sc)

        h = p_ref[0] + p_ref[1]
        h = jnp.maximum(jnp.dot(h, wa_ref[...], preferred_element_type=jnp.float32)
                        + ba_ref[...], 0.0)
        h = jnp.maximum(jnp.dot(h, wb_ref[...], preferred_element_type=jnp.float32)
                        + bb_ref[...], 0.0)
        z_sc[pl.ds(row, BR), :] = h
        st_sc[0:1, :] += jnp.sum(h, axis=0, keepdims=True)
        st_sc[1:2, :] += jnp.sum(h * h, axis=0, keepdims=True)

    @pl.when(ph == 1)
    def _():
        mean = st_sc[0:1, :] * (1.0 / N)
        var = st_sc[1:2, :] * (1.0 / N) - mean * mean
        z = z_sc[pl.ds(row, BR), :]
        o_ref[...] = ((z - mean) * lax.rsqrt(var + EPS_BN) * g_ref[...]
                      + be_ref[...])


_tc_mlp = pl.pallas_call(
    _tc_mlp_body,
    grid=(2, NB),
    in_specs=[
        pl.BlockSpec((2, BR, D), lambda p, b: (0, b * (1 - p), 0)),
        pl.BlockSpec((D, D), lambda p, b: (0, 0)),
        pl.BlockSpec((1, D), lambda p, b: (0, 0)),
        pl.BlockSpec((D, D), lambda p, b: (0, 0)),
        pl.BlockSpec((1, D), lambda p, b: (0, 0)),
        pl.BlockSpec((1, D), lambda p, b: (0, 0)),
        pl.BlockSpec((1, D), lambda p, b: (0, 0)),
    ],
    out_specs=pl.BlockSpec((BR, D), lambda p, b: (b * p, 0)),
    scratch_shapes=[
        pltpu.VMEM((N, D), jnp.float32),
        pltpu.VMEM((8, D), jnp.float32),
    ],
    out_shape=jax.ShapeDtypeStruct((N, D), jnp.float32),
)


def kernel(x, edge_index, W1a, b1a, W1b, b1b, g1, be1, W2a, b2a, W2b, b2b, g2, be2):
    pad = EP - E
    src = jnp.concatenate([edge_index[0], jnp.zeros((pad,), jnp.int32)])
    dst = jnp.concatenate([edge_index[1], jnp.zeros((pad,), jnp.int32)])
    src2 = src.reshape(NW * CPW, CHUNK)
    dst2 = dst.reshape(NW * CPW, CHUNK)
    zeros = jnp.zeros((RPT, D), jnp.float32)

    def layer(h, wa, ba, wb, bb, g, be):
        parts = _sc_agg(h, src2, dst2, zeros).reshape(2, N, D)
        return _tc_mlp(parts, wa, ba.reshape(1, D), wb, bb.reshape(1, D),
                       g.reshape(1, D), be.reshape(1, D))

    h1 = layer(x, W1a, b1a, W1b, b1b, g1, be1)
    return layer(h1, W2a, b2a, W2b, b2b, g2, be2)


# speedup vs baseline: 1.0432x; 1.0432x over previous
import functools

import jax
import jax.numpy as jnp
from jax import lax
from jax.experimental import pallas as pl
from jax.experimental.pallas import tpu as pltpu
from jax.experimental.pallas import tpu_sc as plsc

N = 10000
D = 128
E = 320000
EPS_BN = 1e-5

NC = 2
NS = 16
NW = NC * NS
CHUNK = 128
CPW = 80
NCH = E // CHUNK
S = 16
NST = CPW // S
RPS = S // 2
TMAIN = NCH - 4
CLAMP = TMAIN - S
RPT = 624
TAIL = N - NS * RPT
XSPLIT = 4992

_sc_mesh = plsc.VectorSubcoreMesh(core_axis_name="c", subcore_axis_name="s")


@functools.partial(
    pl.kernel,
    mesh=_sc_mesh,
    out_type=jax.ShapeDtypeStruct((2 * N, D), jnp.float32),
    scratch_types=[
        pltpu.VMEM_SHARED((N, D), jnp.float32),
        pltpu.VMEM((S, CHUNK), jnp.int32),
        pltpu.VMEM((S, CHUNK), jnp.int32),
        pltpu.VMEM((S, CHUNK), jnp.int32),
        pltpu.VMEM((S, CHUNK), jnp.int32),
        pltpu.VMEM((8, CHUNK), jnp.int32),
        pltpu.VMEM((CHUNK, D), jnp.float32),
        pltpu.VMEM((CHUNK, D), jnp.float32),
        pltpu.SemaphoreType.DMA,
        pltpu.SemaphoreType.DMA,
        pltpu.SemaphoreType.DMA,
        pltpu.SemaphoreType.DMA,
        pltpu.SemaphoreType.DMA,
        pltpu.SemaphoreType.DMA,
    ],
)
def _sc_agg(x_hbm, e_hbm, tail_hbm, zero_hbm, out_hbm, acc,
            si0, di0, si1, di1, ti, rb0, rb1, gs0, gs1, ss0, ss1, isrc, idst):
    c = lax.axis_index("c")
    s_ax = lax.axis_index("s")
    wid = c * NS + s_ax
    base = wid * CPW
    row0 = s_ax * RPT

    mine = jnp.where(c == 0, row0 < XSPLIT, row0 >= XSPLIT)

    @pl.when(mine)
    def _():
        pltpu.sync_copy(x_hbm.at[pl.ds(row0, RPT)], acc.at[pl.ds(row0, RPT)])

    @pl.when(jnp.logical_not(mine))
    def _():
        pltpu.sync_copy(zero_hbm.at[pl.ds(0, RPT)], acc.at[pl.ds(row0, RPT)])

    @pl.when(s_ax == NS - 1)
    def _():
        @pl.when(c == 1)
        def _():
            pltpu.sync_copy(x_hbm.at[pl.ds(NS * RPT, TAIL)],
                            acc.at[pl.ds(NS * RPT, TAIL)])

        @pl.when(c == 0)
        def _():
            pltpu.sync_copy(zero_hbm.at[pl.ds(0, TAIL)],
                            acc.at[pl.ds(NS * RPT, TAIL)])

    plsc.subcore_barrier()

    sbuf = (si0, si1)
    dbuf = (di0, di1)

    def stage_off(st):
        return jnp.minimum(base + st * S, CLAMP)

    pltpu.async_copy(e_hbm.at[0, pl.ds(stage_off(0), S)], si0, isrc)
    pltpu.async_copy(e_hbm.at[1, pl.ds(stage_off(0), S)], di0, idst)

    for st in range(NST):
        p = st % 2
        sidx = sbuf[p]
        didx = dbuf[p]
        pltpu.make_async_copy(e_hbm.at[0, pl.ds(stage_off(st), S)], sidx, isrc).wait()
        pltpu.make_async_copy(e_hbm.at[1, pl.ds(stage_off(st), S)], didx, idst).wait()

        def body(j, carry, st=st, p=p, sidx=sidx, didx=didx):
            g0 = base + st * S + 2 * j
            g1 = g0 + 1

            def slot(i_loc, g, rb, gsem, ssem):
                first = (j == 0) if st == 0 else jnp.bool_(False)

                @pl.when(jnp.logical_and(jnp.logical_not(first), g - 2 < TMAIN))
                def _():
                    pltpu.make_async_copy(rb, acc.at[didx.at[i_loc]], ssem).wait()

                @pl.when(g < TMAIN)
                def _():
                    pltpu.async_copy(x_hbm.at[sidx.at[i_loc]], rb, gsem)

            def fire(i_loc, g, rb, gsem, ssem):
                @pl.when(g < TMAIN)
                def _():
                    pltpu.make_async_copy(x_hbm.at[sidx.at[i_loc]], rb, gsem).wait()
                    pltpu.async_copy(rb, acc.at[didx.at[i_loc]], ssem, add=True)

            slot(2 * j, g0, rb0, gs0, ss0)
            slot(2 * j + 1, g1, rb1, gs1, ss1)
            fire(2 * j, g0, rb0, gs0, ss0)
            fire(2 * j + 1, g1, rb1, gs1, ss1)

            if st < NST - 1:
                @pl.when(j == 1)
                def _():
                    pltpu.async_copy(e_hbm.at[0, pl.ds(stage_off(st + 1), S)],
                                     sbuf[1 - p], isrc)
                    pltpu.async_copy(e_hbm.at[1, pl.ds(stage_off(st + 1), S)],
                                     dbuf[1 - p], idst)

            return carry

        lax.fori_loop(0, RPS, body, 0)

    last0 = base + 2 * (CPW // 2 - 1)

    @pl.when(last0 < TMAIN)
    def _():
        pltpu.make_async_copy(rb0, acc.at[dbuf[(NST - 1) % 2].at[0]], ss0).wait()

    @pl.when(last0 + 1 < TMAIN)
    def _():
        pltpu.make_async_copy(rb1, acc.at[dbuf[(NST - 1) % 2].at[1]], ss1).wait()

    @pl.when(wid < 4)
    def _():
        pltpu.sync_copy(tail_hbm, ti)
        pltpu.async_copy(x_hbm.at[ti.at[wid]], rb0, gs0).wait()
        pltpu.sync_copy(rb0, acc.at[ti.at[wid + 4]], add=True)

    plsc.subcore_barrier()
    pltpu.sync_copy(acc.at[pl.ds(row0, RPT)],
                    out_hbm.at[pl.ds(c * N + row0, RPT)])

    @pl.when(s_ax == NS - 1)
    def _():
        pltpu.sync_copy(acc.at[pl.ds(NS * RPT, TAIL)],
                        out_hbm.at[pl.ds(c * N + NS * RPT, TAIL)])


def _tc_mlp_body(p_ref, wa_ref, ba_ref, wb_ref, bb_ref, g_ref, be_ref, o_ref):
    h = p_ref[0:N, :] + p_ref[N:2 * N, :]
    h = jnp.maximum(jnp.dot(h, wa_ref[...], preferred_element_type=jnp.float32)
                    + ba_ref[...], 0.0)
    h = jnp.maximum(jnp.dot(h, wb_ref[...], preferred_element_type=jnp.float32)
                    + bb_ref[...], 0.0)
    mean = jnp.mean(h, axis=0, keepdims=True)
    zc = h - mean
    var = jnp.mean(zc * zc, axis=0, keepdims=True)
    o_ref[...] = zc * lax.rsqrt(var + EPS_BN) * g_ref[...] + be_ref[...]


_tc_mlp = pl.pallas_call(
    _tc_mlp_body,
    out_shape=jax.ShapeDtypeStruct((N, D), jnp.float32),
)


def kernel(x, edge_index, W1a, b1a, W1b, b1b, g1, be1, W2a, b2a, W2b, b2b, g2, be2):
    e3 = edge_index.reshape(2, NCH, CHUNK)
    tail = edge_index[:, TMAIN * CHUNK:].reshape(8, CHUNK)
    zeros = jnp.zeros((RPT, D), jnp.float32)

    def layer(h, wa, ba, wb, bb, g, be):
        parts = _sc_agg(h, e3, tail, zeros)
        return _tc_mlp(parts, wa, ba.reshape(1, D), wb, bb.reshape(1, D),
                       g.reshape(1, D), be.reshape(1, D))

    h1 = layer(x, W1a, b1a, W1b, b1b, g1, be1)
    return layer(h1, W2a, b2a, W2b, b2b, g2, be2)
